# baseline (speedup 1.0000x reference)
import jax
import jax.numpy as jnp
from jax.experimental import pallas as pl
from jax.experimental.pallas import tpu as pltpu

_BM = 512
_BK = 512
_NS = 32
_K = 8192
_NJ = _K // _BK


def _vq_kernel(zsqt_ref, z_ref, tab_hbm, out_ref, tabv_ref, csqv_ref, sems):
    i = pl.program_id(0)
    np_ = _BK // _NS

    @pl.when(i == 0)
    def _start_copies():
        for j in range(_NJ):
            pltpu.make_async_copy(
                tab_hbm.at[pl.ds(j * _BK, _BK), :],
                tabv_ref.at[pl.ds(j * _BK, _BK), :],
                sems.at[j]).start()

    z = z_ref[...]
    z2 = z + z
    zsqt = zsqt_ref[...]

    rmin = jnp.full((_NS, _BM), jnp.inf, dtype=jnp.float32)
    rpk = jnp.zeros((_NS, _BM), dtype=jnp.int32)
    for j in range(_NJ):
        @pl.when(i == 0)
        def _arrive(j=j):
            pltpu.make_async_copy(
                tab_hbm.at[pl.ds(j * _BK, _BK), :],
                tabv_ref.at[pl.ds(j * _BK, _BK), :],
                sems.at[j]).wait()

        tb = tabv_ref[j * _BK:(j + 1) * _BK, :]

        @pl.when(i == 0)
        def _fill_csq(tb=tb, j=j):
            csqv_ref[pl.ds(j * _BK, _BK), :] = jnp.sum(
                tb * tb, axis=1, keepdims=True)

        csq = csqv_ref[j * _BK:(j + 1) * _BK, :]
        cross2 = jax.lax.dot_general(
            tb, z2, (((1,), (1,)), ((), ())),
            preferred_element_type=jnp.float32)
        dt = zsqt - cross2 + csq
        d3 = dt.reshape(np_, _NS, _BM)
        for p in range(np_):
            dq = d3[p]
            upd = dq < rmin
            rmin = jnp.where(upd, dq, rmin)
            rpk = jnp.where(upd, jnp.int32(j * np_ + p), rpk)

    sio = jax.lax.broadcasted_iota(jnp.int32, (_NS, _BM), 0)
    v, k = rmin, rpk * _NS + sio
    s = _NS
    while s > 1:
        sh = s // 2
        va, vb = v[:sh, :], v[sh:s, :]
        ka, kb = k[:sh, :], k[sh:s, :]
        take_b = (vb < va) | ((vb == va) & (kb < ka))
        v = jnp.where(take_b, vb, va)
        k = jnp.where(take_b, kb, ka)
        s = sh
    out_ref[...] = k.reshape(_BM)


def kernel(z_e_x, table):
    B, D = z_e_x.shape
    K, _ = table.shape
    z_sq_t = jnp.sum(z_e_x * z_e_x, axis=-1)[None, :]
    return pl.pallas_call(
        _vq_kernel,
        grid=(B // _BM,),
        in_specs=[
            pl.BlockSpec((1, _BM), lambda i: (0, i)),
            pl.BlockSpec((_BM, D), lambda i: (i, 0)),
            pl.BlockSpec(memory_space=pl.ANY),
        ],
        out_specs=pl.BlockSpec((_BM,), lambda i: (i,)),
        out_shape=jax.ShapeDtypeStruct((B,), jnp.int32),
        scratch_shapes=[
            pltpu.VMEM((K, D), jnp.float32),
            pltpu.VMEM((K, 1), jnp.float32),
            pltpu.SemaphoreType.DMA((_NJ,)),
        ],
    )(z_sq_t, z_e_x, table)


# speedup vs baseline: 1.0167x; 1.0167x over previous
import jax
import jax.numpy as jnp
from jax.experimental import pallas as pl
from jax.experimental.pallas import tpu as pltpu

_BM = 512
_BK = 512
_NS = 32


def _vq_kernel(zsqt_ref, z_ref, tab_ref, out_ref, csqv_ref):
    i = pl.program_id(0)
    K = tab_ref.shape[0]
    num_k = K // _BK
    np_ = _BK // _NS

    z = z_ref[...]
    z2 = z + z
    zsqt = zsqt_ref[...]

    rmin = jnp.full((_NS, _BM), jnp.inf, dtype=jnp.float32)
    rpk = jnp.zeros((_NS, _BM), dtype=jnp.int32)
    for j in range(num_k):
        tb = tab_ref[j * _BK:(j + 1) * _BK, :]

        @pl.when(i == 0)
        def _fill_csq(tb=tb, j=j):
            csqv_ref[pl.ds(j * _BK, _BK), :] = jnp.sum(
                tb * tb, axis=1, keepdims=True)

        csq = csqv_ref[j * _BK:(j + 1) * _BK, :]
        cross2 = jax.lax.dot_general(
            tb, z2, (((1,), (1,)), ((), ())),
            preferred_element_type=jnp.float32)
        dt = zsqt - cross2 + csq
        d3 = dt.reshape(np_, _NS, _BM)
        for p in range(np_):
            dq = d3[p]
            upd = dq < rmin
            rmin = jnp.where(upd, dq, rmin)
            rpk = jnp.where(upd, jnp.int32(j * np_ + p), rpk)

    sio = jax.lax.broadcasted_iota(jnp.int32, (_NS, _BM), 0)
    v, k = rmin, rpk * _NS + sio
    s = _NS
    while s > 1:
        sh = s // 2
        va, vb = v[:sh, :], v[sh:s, :]
        ka, kb = k[:sh, :], k[sh:s, :]
        take_b = (vb < va) | ((vb == va) & (kb < ka))
        v = jnp.where(take_b, vb, va)
        k = jnp.where(take_b, kb, ka)
        s = sh
    out_ref[...] = k.reshape(_BM)


def kernel(z_e_x, table):
    B, D = z_e_x.shape
    K, _ = table.shape
    z_sq_t = jnp.sum(z_e_x * z_e_x, axis=-1)[None, :]
    return pl.pallas_call(
        _vq_kernel,
        grid=(B // _BM,),
        in_specs=[
            pl.BlockSpec((1, _BM), lambda i: (0, i)),
            pl.BlockSpec((_BM, D), lambda i: (i, 0)),
            pl.BlockSpec((K, D), lambda i: (0, 0)),
        ],
        out_specs=pl.BlockSpec((_BM,), lambda i: (i,)),
        out_shape=jax.ShapeDtypeStruct((B,), jnp.int32),
        scratch_shapes=[
            pltpu.VMEM((K, 1), jnp.float32),
        ],
    )(z_sq_t, z_e_x, table)
